# baseline (speedup 1.0000x reference)
import functools

import jax
import jax.numpy as jnp
from jax import lax
from jax.experimental import pallas as pl
from jax.experimental.pallas import tpu as pltpu
from jax.experimental.pallas import tpu_sc as plsc

BATCH = 16384
N_PERM = 200
N_EMBED = 64

ROWS_PER_CHUNK = 2
CHUNK = ROWS_PER_CHUNK * N_PERM
VREGS = CHUNK // 16
GATHER_SIZES = (128, 128, 128, 16)


def _build_table_tc(c_perm, pos_embedding):
    def body(pos_ref, cp_ref, out_ref):
        blk = pos_ref[...][:, None, :] + cp_ref[...][None, :, :]
        out_ref[...] = blk.reshape(8 * N_PERM, N_EMBED)

    return pl.pallas_call(
        body,
        grid=(N_PERM // 8,),
        in_specs=[
            pl.BlockSpec((8, N_EMBED), lambda i: (i, 0)),
            pl.BlockSpec((N_PERM, N_EMBED), lambda i: (0, 0)),
        ],
        out_specs=pl.BlockSpec((8 * N_PERM, N_EMBED), lambda i: (i, 0)),
        out_shape=jax.ShapeDtypeStruct((N_PERM * N_PERM, N_EMBED), jnp.float32),
    )(pos_embedding, c_perm)


def _sc_gather(x, table_flat):
    info = plsc.get_sparse_core_info()
    nw = info.num_cores * info.num_subcores
    rows_w = BATCH // nw
    n_chunks = rows_w // ROWS_PER_CHUNK

    mesh = plsc.VectorSubcoreMesh(core_axis_name="c", subcore_axis_name="s")

    @functools.partial(
        pl.kernel,
        out_type=jax.ShapeDtypeStruct((BATCH, N_PERM, N_EMBED), jnp.float32),
        mesh=mesh,
        scratch_types=[
            pltpu.VMEM((2, CHUNK), jnp.int32),
            pltpu.VMEM((2, 4, 128), jnp.int32),
            pltpu.VMEM((2, CHUNK, N_EMBED), jnp.float32),
            pltpu.SemaphoreType.DMA,
            pltpu.SemaphoreType.DMA,
            pltpu.SemaphoreType.DMA,
            pltpu.SemaphoreType.DMA,
        ],
        compiler_params=pltpu.CompilerParams(use_tc_tiling_on_sc=False),
    )
    def k(x_hbm, t_hbm, out_hbm, idx_v, flat_v, rows_v,
          sem_g0, sem_g1, sem_o0, sem_o1):
        wid = lax.axis_index("s") * info.num_cores + lax.axis_index("c")
        wrow = wid * rows_w
        sem_g = (sem_g0, sem_g1)
        sem_o = (sem_o0, sem_o1)
        iota = lax.iota(jnp.int32, 16)

        def stage_indices(t, b):
            r0 = wrow + t * ROWS_PER_CHUNK
            for r in range(ROWS_PER_CHUNK):
                pltpu.sync_copy(x_hbm.at[r0 + r],
                                idx_v.at[b].at[pl.ds(r * N_PERM, N_PERM)])
            for k16 in range(VREGS):
                xv = idx_v[b, pl.ds(k16 * 16, 16)]
                p = lax.rem(iota + (k16 * 16), N_PERM)
                flat_v[b, k16 // 8, pl.ds((k16 % 8) * 16, 16)] = p * N_PERM + xv

        def gather_copies(b):
            off = 0
            copies = []
            for j, g in enumerate(GATHER_SIZES):
                copies.append(pltpu.make_async_copy(
                    t_hbm.at[flat_v.at[b].at[j].at[pl.ds(0, g)]],
                    rows_v.at[b].at[pl.ds(off, g)],
                    sem_g[b],
                ))
                off += g
            return copies

        def fire_gathers(b):
            for c in gather_copies(b):
                c.start()

        def wait_gathers(b):
            for c in gather_copies(b):
                c.wait()

        def out_copies(t, b):
            r0 = wrow + t * ROWS_PER_CHUNK
            return [
                pltpu.make_async_copy(
                    rows_v.at[b].at[pl.ds(r * N_PERM, N_PERM)],
                    out_hbm.at[r0 + r],
                    sem_o[b],
                )
                for r in range(ROWS_PER_CHUNK)
            ]

        def fire_outcopy(t, b):
            for c in out_copies(t, b):
                c.start()

        def wait_outcopy(b):
            for c in out_copies(0, b):
                c.wait()

        stage_indices(0, 0)
        fire_gathers(0)

        def outer(g, carry):
            for b in (0, 1):
                t = 2 * g + b + 1
                bb = (b + 1) % 2

                @pl.when(t < n_chunks)
                def _():
                    stage_indices(t, bb)

                    @pl.when(t >= 2)
                    def _():
                        wait_outcopy(bb)

                    fire_gathers(bb)
                wait_gathers(b)
                fire_outcopy(t - 1, b)
            return carry

        lax.fori_loop(0, (n_chunks + 1) // 2, outer, 0)
        wait_outcopy(0)
        wait_outcopy(1)

    return k(x, table_flat)


def kernel(x, c_perm, pos_embedding):
    table_flat = _build_table_tc(c_perm, pos_embedding)
    return _sc_gather(x.astype(jnp.int32), table_flat)


# speedup vs baseline: 1.0952x; 1.0952x over previous
import functools

import jax
import jax.numpy as jnp
from jax import lax
from jax.experimental import pallas as pl
from jax.experimental.pallas import tpu as pltpu
from jax.experimental.pallas import tpu_sc as plsc

BATCH = 16384
N_PERM = 200
N_EMBED = 64
LANES = 128

ROWS_PER_CHUNK = 1
CHUNK = ROWS_PER_CHUNK * N_PERM
VREGS = 13
XFETCH = 256
GATHER_SIZES = (128, 72)


def _build_table_tc(c_perm, pos_embedding):
    def body(pos_ref, cp_ref, out_ref):
        blk = pos_ref[...][:, None, :] + cp_ref[...][None, :, :]
        blk = blk.reshape(8 * N_PERM, N_EMBED)
        out_ref[...] = jnp.concatenate([blk, blk], axis=-1)

    return pl.pallas_call(
        body,
        grid=(N_PERM // 8,),
        in_specs=[
            pl.BlockSpec((8, N_EMBED), lambda i: (i, 0)),
            pl.BlockSpec((N_PERM, N_EMBED), lambda i: (0, 0)),
        ],
        out_specs=pl.BlockSpec((8 * N_PERM, LANES), lambda i: (i, 0)),
        out_shape=jax.ShapeDtypeStruct((N_PERM * N_PERM, LANES), jnp.float32),
    )(pos_embedding, c_perm)


def _sc_gather(x_flat, table):
    info = plsc.get_sparse_core_info()
    nw = info.num_cores * info.num_subcores
    rows_w = BATCH // nw
    n_chunks = rows_w // ROWS_PER_CHUNK

    mesh = plsc.VectorSubcoreMesh(core_axis_name="c", subcore_axis_name="s")

    @functools.partial(
        pl.kernel,
        out_type=jax.ShapeDtypeStruct((BATCH, N_PERM, N_EMBED), jnp.float32),
        mesh=mesh,
        scratch_types=[
            pltpu.VMEM((XFETCH,), jnp.int32),
            pltpu.VMEM((XFETCH,), jnp.int32),
            pltpu.VMEM((2, 8, 128), jnp.int32),
            pltpu.VMEM((2, CHUNK, LANES), jnp.float32),
            pltpu.VMEM((2, CHUNK, N_EMBED), jnp.float32),
            pltpu.SemaphoreType.DMA,
            pltpu.SemaphoreType.DMA,
            pltpu.SemaphoreType.DMA,
            pltpu.SemaphoreType.DMA,
        ],
    )
    def k(x_hbm, t_hbm, out_hbm, idx_v0, idx_v1, flat_v, rows_v, comp_v,
          sem_g0, sem_g1, sem_o0, sem_o1):
        wid = lax.axis_index("s") * info.num_cores + lax.axis_index("c")
        wrow = wid * rows_w
        idx_v = (idx_v0, idx_v1)
        sem_g = (sem_g0, sem_g1)
        sem_o = (sem_o0, sem_o1)
        iota = lax.iota(jnp.int32, 16)

        def stage_indices(t, b):
            base = (wrow + t * ROWS_PER_CHUNK) * N_PERM
            pltpu.sync_copy(x_hbm.at[pl.ds(base, XFETCH)], idx_v[b])
            for k16 in range(VREGS):
                xv = idx_v[b][pl.ds(k16 * 16, 16)]
                p = lax.rem(iota + (k16 * 16), N_PERM)
                flat_v[b, k16 // 8, pl.ds((k16 % 8) * 16, 16)] = p * N_PERM + xv

        def gather_copies(b):
            off = 0
            copies = []
            for j, g in enumerate(GATHER_SIZES):
                copies.append(pltpu.make_async_copy(
                    t_hbm.at[flat_v.at[b].at[j].at[pl.ds(0, g)]],
                    rows_v.at[b].at[pl.ds(off, g)],
                    sem_g[b],
                ))
                off += g
            return copies

        def fire_gathers(b):
            for c in gather_copies(b):
                c.start()

        def wait_gathers(b):
            for c in gather_copies(b):
                c.wait()

        def compact(b):
            unroll = 8

            def body(i, carry):
                row = i * unroll
                for u in range(unroll):
                    for c in range(N_EMBED // 16):
                        comp_v[b, row + u, pl.ds(c * 16, 16)] = (
                            rows_v[b, row + u, pl.ds(c * 16, 16)])
                return carry

            lax.fori_loop(0, CHUNK // unroll, body, 0)

        def out_copies(t, b):
            r0 = wrow + t * ROWS_PER_CHUNK
            return [
                pltpu.make_async_copy(
                    comp_v.at[b, pl.ds(r * N_PERM, N_PERM), pl.ds(0, N_EMBED)],
                    out_hbm.at[r0 + r],
                    sem_o[b],
                )
                for r in range(ROWS_PER_CHUNK)
            ]

        def fire_outcopy(t, b):
            for c in out_copies(t, b):
                c.start()

        def wait_outcopy(b):
            for c in out_copies(0, b):
                c.wait()

        stage_indices(0, 0)
        fire_gathers(0)

        def outer(g, carry):
            for b in (0, 1):
                t = 2 * g + b + 1
                bb = (b + 1) % 2

                @pl.when(t < n_chunks)
                def _():
                    stage_indices(t, bb)

                    @pl.when(t >= 2)
                    def _():
                        wait_outcopy(bb)

                    fire_gathers(bb)
                wait_gathers(b)
                compact(b)
                fire_outcopy(t - 1, b)
            return carry

        lax.fori_loop(0, (n_chunks + 1) // 2, outer, 0)
        wait_outcopy(0)
        wait_outcopy(1)

    return k(x_flat, table)


def kernel(x, c_perm, pos_embedding):
    table = _build_table_tc(c_perm, pos_embedding)
    x_flat = jnp.concatenate(
        [x.reshape(BATCH * N_PERM).astype(jnp.int32),
         jnp.zeros((128,), jnp.int32)])
    return _sc_gather(x_flat, table)
